# baseline (speedup 1.0000x reference)
import functools

import jax
import jax.numpy as jnp
from jax import lax
from jax.experimental import pallas as pl
from jax.experimental.pallas import tpu as pltpu
from jax.experimental.pallas import tpu_sc as plsc

_B = 16384
_D = 64
_NC = 2
_NS = 16
_NW = _NC * _NS
_BPW = _B // _NW
_L = 16
_V = 1000000
_BLK = 16384
_SH = _BLK.bit_length() - 1
_HMASK = _BLK // 2 - 1
_GRID = (_V + _BLK - 1) // _BLK
_YROWS = _GRID * (_BLK // 2)


def _retile_body(u_ref, m_ref, yu_ref, ym_ref):
    xu = u_ref[...].T
    yu_ref[...] = jnp.concatenate([xu[: _BLK // 2], xu[_BLK // 2:]], axis=1)
    xm = m_ref[...].T
    ym_ref[...] = jnp.concatenate([xm[: _BLK // 2], xm[_BLK // 2:]], axis=1)


def _retile(ut, mt):
    return pl.pallas_call(
        _retile_body,
        grid=(_GRID,),
        in_specs=[
            pl.BlockSpec((_D, _BLK), lambda i: (0, i)),
            pl.BlockSpec((_D, _BLK), lambda i: (0, i)),
        ],
        out_specs=[
            pl.BlockSpec((_BLK // 2, 2 * _D), lambda i: (i, 0)),
            pl.BlockSpec((_BLK // 2, 2 * _D), lambda i: (i, 0)),
        ],
        out_shape=[
            jax.ShapeDtypeStruct((_YROWS, 2 * _D), jnp.float32),
            jax.ShapeDtypeStruct((_YROWS, 2 * _D), jnp.float32),
        ],
        compiler_params=pltpu.CompilerParams(
            dimension_semantics=("arbitrary",),
            vmem_limit_bytes=100 * 1024 * 1024),
    )(ut, mt)


def _mf_body(uid_hbm, mid_hbm, yu_hbm, ym_hbm, ubias_hbm, mbias_hbm,
             out_hbm,
             uid_v, mid_v, row_v, big_v, uc_v, ubias_v, mbias_v, out_v,
             sem_g, sem_b):
    wid = lax.axis_index("s") * _NC + lax.axis_index("c")
    base = wid * _BPW

    pltpu.sync_copy(uid_hbm.at[pl.ds(base, _BPW)], uid_v)
    pltpu.sync_copy(mid_hbm.at[pl.ds(base, _BPW)], mid_v)

    cbu = pltpu.async_copy(ubias_hbm.at[uid_v], ubias_v, sem_b)
    cbm = pltpu.async_copy(mbias_hbm.at[mid_v], mbias_v, sem_b)

    def packed_rows(ids_v, _unused):
        def chunk(c, carry):
            v = ids_v[pl.ds(c * _L, _L)]
            row_v[pl.ds(c * _L, _L)] = v >> 1
            return carry
        lax.fori_loop(0, _BPW // _L, chunk, 0)

    packed_rows(uid_v, 1)
    pltpu.async_copy(yu_hbm.at[row_v], big_v, sem_g).wait()

    def unpack_group(g, carry):
        r0 = g * _L
        rows = lax.iota(jnp.int32, _L) + r0
        colbase = (uid_v[pl.ds(r0, _L)] & 1) * _D
        flat = rows * _D
        for d in range(_D):
            vals = plsc.load_gather(big_v, [rows, colbase + d])
            plsc.store_scatter(uc_v, [flat + d], vals)
        return carry

    lax.fori_loop(0, _BPW // _L, unpack_group, 0)

    packed_rows(mid_v, 1)
    pltpu.async_copy(ym_hbm.at[row_v], big_v, sem_g).wait()

    cbu.wait()
    cbm.wait()

    def dot_group(g, carry):
        r0 = g * _L
        rows = lax.iota(jnp.int32, _L) + r0
        colbase = (mid_v[pl.ds(r0, _L)] & 1) * _D
        flat = rows * _D
        acc = ubias_v[pl.ds(r0, _L)] + mbias_v[pl.ds(r0, _L)]
        for d in range(_D):
            u = plsc.load_gather(uc_v, [flat + d])
            m = plsc.load_gather(big_v, [rows, colbase + d])
            acc = acc + u * m
        out_v[pl.ds(r0, _L)] = acc
        return carry

    lax.fori_loop(0, _BPW // _L, dot_group, 0)
    pltpu.sync_copy(out_v, out_hbm.at[pl.ds(base, _BPW)])


@functools.partial(jax.jit, donate_argnums=())
def kernel(user_ids, movie_ids, user_emb, movie_emb, user_bias, movie_bias):
    yu = user_emb.reshape(_V // 2, 2 * _D)
    ym = movie_emb.reshape(_V // 2, 2 * _D)
    run = pl.kernel(
        _mf_body,
        out_type=jax.ShapeDtypeStruct((_B,), jnp.float32),
        mesh=plsc.VectorSubcoreMesh(core_axis_name="c", subcore_axis_name="s"),
        compiler_params=pltpu.CompilerParams(
            needs_layout_passes=False, use_tc_tiling_on_sc=False),
        scratch_types=[
            pltpu.VMEM((_BPW,), jnp.int32),
            pltpu.VMEM((_BPW,), jnp.int32),
            pltpu.VMEM((_BPW,), jnp.int32),
            pltpu.VMEM((_BPW, 2 * _D), jnp.float32),
            pltpu.VMEM((_BPW * _D,), jnp.float32),
            pltpu.VMEM((_BPW,), jnp.float32),
            pltpu.VMEM((_BPW,), jnp.float32),
            pltpu.VMEM((_BPW,), jnp.float32),
            pltpu.SemaphoreType.DMA,
            pltpu.SemaphoreType.DMA,
        ],
    )
    return run(user_ids.astype(jnp.int32), movie_ids.astype(jnp.int32),
               yu, ym,
               user_bias.reshape(-1), movie_bias.reshape(-1))


# speedup vs baseline: 2.1240x; 2.1240x over previous
import functools

import jax
import jax.numpy as jnp
from jax import lax
from jax.experimental import pallas as pl
from jax.experimental.pallas import tpu as pltpu
from jax.experimental.pallas import tpu_sc as plsc

_B = 16384
_D = 64
_NC = 2
_NS = 16
_NW = _NC * _NS
_BPW = _B // _NW
_L = 16
_V = 1000000
_BLK = 16384
_SH = _BLK.bit_length() - 1
_HMASK = _BLK // 2 - 1
_GRID = (_V + _BLK - 1) // _BLK
_YROWS = _GRID * (_BLK // 2)


def _retile_body(u_ref, m_ref, yu_ref, ym_ref):
    xu = u_ref[...].T
    yu_ref[...] = jnp.concatenate([xu[: _BLK // 2], xu[_BLK // 2:]], axis=1)
    xm = m_ref[...].T
    ym_ref[...] = jnp.concatenate([xm[: _BLK // 2], xm[_BLK // 2:]], axis=1)


def _retile(ut, mt):
    return pl.pallas_call(
        _retile_body,
        grid=(_GRID,),
        in_specs=[
            pl.BlockSpec((_D, _BLK), lambda i: (0, i)),
            pl.BlockSpec((_D, _BLK), lambda i: (0, i)),
        ],
        out_specs=[
            pl.BlockSpec((_BLK // 2, 2 * _D), lambda i: (i, 0)),
            pl.BlockSpec((_BLK // 2, 2 * _D), lambda i: (i, 0)),
        ],
        out_shape=[
            jax.ShapeDtypeStruct((_YROWS, 2 * _D), jnp.float32),
            jax.ShapeDtypeStruct((_YROWS, 2 * _D), jnp.float32),
        ],
        compiler_params=pltpu.CompilerParams(
            dimension_semantics=("arbitrary",),
            vmem_limit_bytes=100 * 1024 * 1024),
    )(ut, mt)


def _mf_body(uid_hbm, mid_hbm, yu_hbm, ym_hbm, ubias_hbm, mbias_hbm,
             out_hbm,
             uid_v, mid_v, rowu0_v, rowu1_v, rowm0_v, rowm1_v,
             bu_v, bm_v, ubias_v, mbias_v, out_v,
             sem_g, sem_b):
    wid = lax.axis_index("s") * _NC + lax.axis_index("c")
    base = wid * _BPW
    half = _BPW // 2

    pltpu.sync_copy(uid_hbm.at[pl.ds(base, _BPW)], uid_v)
    pltpu.sync_copy(mid_hbm.at[pl.ds(base, _BPW)], mid_v)

    cbu = pltpu.async_copy(ubias_hbm.at[uid_v], ubias_v, sem_b)
    cbm = pltpu.async_copy(mbias_hbm.at[mid_v], mbias_v, sem_b)

    def packed_rows(ids_v, dst, off):
        def chunk(c, carry):
            v = ids_v[pl.ds(off + c * _L, _L)]
            dst[pl.ds(c * _L, _L)] = ((v >> _SH) << (_SH - 1)) + (v & _HMASK)
            return carry
        lax.fori_loop(0, half // _L, chunk, 0)

    packed_rows(uid_v, rowu0_v, 0)
    packed_rows(uid_v, rowu1_v, half)
    packed_rows(mid_v, rowm0_v, 0)
    packed_rows(mid_v, rowm1_v, half)

    cbu.wait()
    cbm.wait()

    for h, (ru, rm) in enumerate(((rowu0_v, rowm0_v), (rowu1_v, rowm1_v))):
        off = h * half
        cu = pltpu.async_copy(yu_hbm.at[ru], bu_v, sem_g)
        cm = pltpu.async_copy(ym_hbm.at[rm], bm_v, sem_g)
        cu.wait()
        cm.wait()

        def dot_group(g, carry):
            r0 = g * _L
            rows = lax.iota(jnp.int32, _L) + r0
            cbase_u = ((uid_v[pl.ds(off + r0, _L)] >> (_SH - 1)) & 1) * _D
            cbase_m = ((mid_v[pl.ds(off + r0, _L)] >> (_SH - 1)) & 1) * _D
            acc = (ubias_v[pl.ds(off + r0, _L)] +
                   mbias_v[pl.ds(off + r0, _L)])
            for d in range(_D):
                u = plsc.load_gather(bu_v, [rows, cbase_u + d])
                m = plsc.load_gather(bm_v, [rows, cbase_m + d])
                acc = acc + u * m
            out_v[pl.ds(off + r0, _L)] = acc
            return carry

        lax.fori_loop(0, half // _L, dot_group, 0)

    pltpu.sync_copy(out_v, out_hbm.at[pl.ds(base, _BPW)])


@functools.partial(jax.jit, donate_argnums=())
def kernel(user_ids, movie_ids, user_emb, movie_emb, user_bias, movie_bias):
    yu, ym = _retile(user_emb.T, movie_emb.T)
    run = pl.kernel(
        _mf_body,
        out_type=jax.ShapeDtypeStruct((_B,), jnp.float32),
        mesh=plsc.VectorSubcoreMesh(core_axis_name="c", subcore_axis_name="s"),
        compiler_params=pltpu.CompilerParams(
            needs_layout_passes=False, use_tc_tiling_on_sc=False),
        scratch_types=[
            pltpu.VMEM((_BPW,), jnp.int32),
            pltpu.VMEM((_BPW,), jnp.int32),
            pltpu.VMEM((_BPW // 2,), jnp.int32),
            pltpu.VMEM((_BPW // 2,), jnp.int32),
            pltpu.VMEM((_BPW // 2,), jnp.int32),
            pltpu.VMEM((_BPW // 2,), jnp.int32),
            pltpu.VMEM((_BPW // 2, 2 * _D), jnp.float32),
            pltpu.VMEM((_BPW // 2, 2 * _D), jnp.float32),
            pltpu.VMEM((_BPW,), jnp.float32),
            pltpu.VMEM((_BPW,), jnp.float32),
            pltpu.VMEM((_BPW,), jnp.float32),
            pltpu.SemaphoreType.DMA,
            pltpu.SemaphoreType.DMA,
        ],
    )
    return run(user_ids.astype(jnp.int32), movie_ids.astype(jnp.int32),
               yu, ym,
               user_bias.reshape(-1), movie_bias.reshape(-1))
